# baseline (speedup 1.0000x reference)
import functools

import jax
import jax.numpy as jnp
from jax import lax
from jax.experimental import pallas as pl
from jax.experimental.pallas import tpu as pltpu
from jax.experimental.pallas import tpu_sc as plsc

B = 4
N = 4096
STRIDE = 4
RADIUS = 0.2
K = 32
IN_C = 64
OUT_C = 128
EPS = 1e-5
M = N // STRIDE
R = B * M * K

NC, NS, L = 2, 16, 16
NW = NC * NS
CPW = (B * M) // NW
RPW = R // NW
GCH = 128


_FR = 8
_FC = N // _FR
_OC = 128
_OR = M // _OC


def _fps_body(xs_ref, ys_ref, zs_ref, p3_ref, ox_ref, oy_ref, oz_ref):
    nidx = (lax.broadcasted_iota(jnp.int32, (_FR, _FC), 0) * _FC
            + lax.broadcasted_iota(jnp.int32, (_FR, _FC), 1))
    orow = lax.broadcasted_iota(jnp.int32, (_OR, _OC), 0)
    ocol = lax.broadcasted_iota(jnp.int32, (_OR, _OC), 1)

    Xs = [xs_ref[b] for b in range(B)]
    Ys = [ys_ref[b] for b in range(B)]
    Zs = [zs_ref[b] for b in range(B)]
    Ps = [p3_ref[b] for b in range(B)]

    sel0 = (orow == 0) & (ocol == 0)
    st0 = []
    for b in range(B):
        cx = Xs[b][0:1, 0:1]
        cy = Ys[b][0:1, 0:1]
        cz = Zs[b][0:1, 0:1]
        dis0 = jnp.full((_FR, _FC), 1e10, jnp.float32)
        st0.extend([dis0, cx, cy, cz,
                    jnp.where(sel0, cx, 0.0),
                    jnp.where(sel0, cy, 0.0),
                    jnp.where(sel0, cz, 0.0)])

    def body(i, carry):
        st = list(carry)
        sel = (orow == i // _OC) & (ocol == i % _OC)
        for b in range(B):
            dis, cx, cy, cz, ax, ay, az = st[7 * b:7 * b + 7]
            d = (Xs[b] - cx) ** 2 + (Ys[b] - cy) ** 2 + (Zs[b] - cz) ** 2
            dis = jnp.minimum(dis, d)
            v = jnp.max(jnp.max(dis, axis=1, keepdims=True), axis=0,
                        keepdims=True)
            cand = jnp.where(dis == v, nidx, N)
            idx = jnp.min(jnp.min(cand, axis=1, keepdims=True), axis=0,
                          keepdims=True)
            m1 = nidx == idx
            m3 = jnp.broadcast_to(m1[None], (3, _FR, _FC))
            s3 = jnp.sum(jnp.where(m3, Ps[b], 0.0), axis=(1, 2),
                         keepdims=True)
            ncx = s3[0]
            ncy = s3[1]
            ncz = s3[2]
            st[7 * b:7 * b + 7] = [dis, ncx, ncy, ncz,
                                   jnp.where(sel, ncx, ax),
                                   jnp.where(sel, ncy, ay),
                                   jnp.where(sel, ncz, az)]
        return tuple(st)

    st = lax.fori_loop(1, M, body, tuple(st0))
    for b in range(B):
        ox_ref[b] = st[7 * b + 4]
        oy_ref[b] = st[7 * b + 5]
        oz_ref[b] = st[7 * b + 6]


def _fps_call(xs, ys, zs):
    p3 = jnp.stack([xs, ys, zs], axis=1)
    return pl.pallas_call(
        _fps_body,
        out_shape=[jax.ShapeDtypeStruct((B, _OR, _OC), jnp.float32)] * 3,
    )(xs, ys, zs, p3)


def _xstats_body(f2_ref, xw_ref, g_ref, b_ref, scale_ref, shift_ref):
    F = f2_ref[...]
    W = xw_ref[...]
    cnt = float(B * N)
    G = lax.dot_general(F, F, (((0,), (0,)), ((), ())),
                        preferred_element_type=jnp.float32) / cnt
    mu = jnp.sum(F, axis=0, keepdims=True) / cnt
    mean = lax.dot_general(W, mu, (((1,), (1,)), ((), ())))
    T = lax.dot_general(W, G, (((1,), (0,)), ((), ())),
                        preferred_element_type=jnp.float32)
    ey2 = jnp.sum(T * W, axis=1, keepdims=True)
    var = ey2 - mean * mean
    sc = g_ref[...] / jnp.sqrt(var + EPS)
    scale_ref[...] = sc
    shift_ref[...] = b_ref[...] - mean * sc


def _xstats_call(f2, x_w, gamma, beta):
    return pl.pallas_call(
        _xstats_body,
        out_shape=[jax.ShapeDtypeStruct((OUT_C, 1), jnp.float32)] * 2,
    )(f2, x_w, gamma, beta)


def _feats_body(f2_ref, wt_ref, scale_ref, shift_ref, out_ref):
    y = lax.dot_general(f2_ref[...], wt_ref[...], (((1,), (0,)), ((), ())),
                        preferred_element_type=jnp.float32)
    out_ref[...] = jnp.maximum(y * scale_ref[...] + shift_ref[...], 0.0)


def _feats_call(f2, x_wt, scale, shift):
    blk = 2048
    grid = (B * N) // blk
    return pl.pallas_call(
        _feats_body,
        grid=(grid,),
        in_specs=[
            pl.BlockSpec((blk, IN_C), lambda i: (i, 0)),
            pl.BlockSpec((IN_C, OUT_C), lambda i: (0, 0)),
            pl.BlockSpec((1, OUT_C), lambda i: (0, 0)),
            pl.BlockSpec((1, OUT_C), lambda i: (0, 0)),
        ],
        out_specs=pl.BlockSpec((blk, OUT_C), lambda i: (i, 0)),
        out_shape=jax.ShapeDtypeStruct((B * N, OUT_C), jnp.float32),
    )(f2, x_wt, scale, shift)


def _ballquery_call(cx, cy, cz, px, py, pz, feats):
    mesh = plsc.VectorSubcoreMesh(core_axis_name="c", subcore_axis_name="s")

    @functools.partial(
        pl.kernel,
        mesh=mesh,
        compiler_params=pltpu.CompilerParams(needs_layout_passes=False),
        out_type=[
            jax.ShapeDtypeStruct((R * 4,), jnp.float32),
            jax.ShapeDtypeStruct((R, OUT_C), jnp.float32),
        ],
        scratch_types=[
            pltpu.VMEM((N,), jnp.float32),
            pltpu.VMEM((N,), jnp.float32),
            pltpu.VMEM((N,), jnp.float32),
            pltpu.VMEM((CPW,), jnp.float32),
            pltpu.VMEM((CPW,), jnp.float32),
            pltpu.VMEM((CPW,), jnp.float32),
            pltpu.VMEM((256,), jnp.int32),
            pltpu.VMEM((CPW * K,), jnp.int32),
            pltpu.VMEM((CPW * K * 4,), jnp.float32),
            pltpu.VMEM((GCH, OUT_C), jnp.float32),
            pltpu.VMEM((GCH, OUT_C), jnp.float32),
            pltpu.SemaphoreType.DMA,
            pltpu.SemaphoreType.DMA,
        ],
    )
    def k(cxh, cyh, czh, pxh, pyh, pzh, fh, gp4h, oh,
          pxv, pyv, pzv, cxv, cyv, czv, hitv, idxv, gpv,
          bv0, bv1, sem0, sem1):
        wid = lax.axis_index("s") * NC + lax.axis_index("c")
        g0 = wid * CPW
        bi = g0 // M
        off = g0 - bi * M
        pltpu.sync_copy(pxh.at[bi], pxv)
        pltpu.sync_copy(pyh.at[bi], pyv)
        pltpu.sync_copy(pzh.at[bi], pzv)
        pltpu.sync_copy(cxh.at[bi, pl.ds(off, CPW)], cxv)
        pltpu.sync_copy(cyh.at[bi, pl.ds(off, CPW)], cyv)
        pltpu.sync_copy(czh.at[bi, pl.ds(off, CPW)], czv)

        lane = lax.iota(jnp.int32, 16)
        zero16 = jnp.zeros((16,), jnp.int32)
        col0 = zero16
        col1 = zero16 + 1
        col2 = zero16 + 2
        col3 = zero16 + 3
        zf16 = jnp.zeros((16,), jnp.float32)
        r2 = RADIUS * RADIUS
        gbase = bi * N

        czero = jnp.zeros((16,), jnp.int32)

        NG = 4

        def per_quad(cp, _):
            ia = jnp.full((16,), NG * cp, jnp.int32)
            cents = [(plsc.load_gather(cxv, [ia + t]),
                      plsc.load_gather(cyv, [ia + t]),
                      plsc.load_gather(czv, [ia + t])) for t in range(NG)]

            def cond(st):
                j = st[0]
                cs = st[1:]
                lo = jnp.minimum(jnp.minimum(cs[0], cs[1]),
                                 jnp.minimum(cs[2], cs[3]))
                return (j < N // 16) & (jnp.max(lo) < K)

            def body(st):
                j = st[0]
                cnts = list(st[1:])
                for u in range(2):
                    base = (j + u) * 16
                    x16 = pxv[pl.ds(base, 16)]
                    y16 = pyv[pl.ds(base, 16)]
                    z16 = pzv[pl.ds(base, 16)]
                    gidx = base + lane
                    for t in range(NG):
                        cxs, cys, czs = cents[t]
                        dx = x16 - cxs
                        dy = y16 - cys
                        dz = z16 - czs
                        d2 = dx * dx + dy * dy + dz * dz
                        msk = d2 <= r2
                        mi = msk.astype(jnp.int32)
                        pos = cnts[t] + plsc.cumsum(mi) - mi
                        ok = msk & (pos < 64)
                        plsc.store_scatter(hitv, [pos + 64 * t], gidx, mask=ok)
                        cnts[t] = cnts[t] + plsc.all_reduce_population_count(msk)
                return (j + 2, *cnts)

            st = lax.while_loop(cond, body, (0, czero, czero, czero, czero))
            for t in range(NG):
                cxs, cys, czs = cents[t]
                cnt = jnp.minimum(jnp.max(st[1 + t]), K)
                hb = 64 * t
                first = plsc.load_gather(hitv, [jnp.full((16,), hb, jnp.int32)])
                o0 = jnp.where(lane < cnt, hitv[pl.ds(hb, 16)], first)
                o1 = jnp.where(lane + 16 < cnt, hitv[pl.ds(hb + 16, 16)], first)
                c = NG * cp + t
                rows0 = c * K + lane
                rows1 = rows0 + 16
                for o, rows in ((o0, rows0), (o1, rows1)):
                    gx = (plsc.load_gather(pxv, [o]) - cxs) / RADIUS
                    gy = (plsc.load_gather(pyv, [o]) - cys) / RADIUS
                    gz = (plsc.load_gather(pzv, [o]) - czs) / RADIUS
                    flat = rows * 4
                    plsc.store_scatter(gpv, [flat], gx)
                    plsc.store_scatter(gpv, [flat + 1], gy)
                    plsc.store_scatter(gpv, [flat + 2], gz)
                    plsc.store_scatter(gpv, [flat + 3], zf16)
                idxv[pl.ds(c * K, 16)] = o0 + gbase
                idxv[pl.ds(c * K + 16, 16)] = o1 + gbase
            return 0

        lax.fori_loop(0, CPW // NG, per_quad, 0)
        pltpu.sync_copy(gpv, gp4h.at[pl.ds(g0 * K * 4, CPW * K * 4)])

        nch = (CPW * K) // GCH
        bvs = (bv0, bv1)
        sems = (sem0, sem1)
        pltpu.async_copy(fh.at[idxv.at[pl.ds(0, GCH)]], bv0, sem0)

        def gouter(g2, _):
            g = g2 * 2
            for par in range(2):
                t = g + par
                nxt = t + 1
                np_ = (par + 1) % 2

                @pl.when(nxt < nch)
                def _():
                    pltpu.async_copy(fh.at[idxv.at[pl.ds(nxt * GCH, GCH)]],
                                     bvs[np_], sems[np_])

                pltpu.make_async_copy(fh.at[idxv.at[pl.ds(t * GCH, GCH)]],
                                      bvs[par], sems[par]).wait()
                pltpu.sync_copy(bvs[par],
                                oh.at[pl.ds(g0 * K + t * GCH, GCH)])
            return 0

        lax.fori_loop(0, nch // 2, gouter, 0)

    return k(cx, cy, cz, px, py, pz, feats)


def _pstats_body(gpq_ref, w4_ref, g_ref, b_ref, pscale_ref, pshift_ref):
    A = gpq_ref[...]
    W4 = w4_ref[...]
    cnt = float(R)
    row = lax.broadcasted_iota(jnp.int32, (128, K), 0)
    col = lax.broadcasted_iota(jnp.int32, (128, K), 1)
    comps = []
    for c in range(3):
        S = (row == col * 4 + c).astype(jnp.float32)
        comps.append(lax.dot_general(A, S, (((1,), (0,)), ((), ())),
                                     preferred_element_type=jnp.float32))
    gx, gy, gz = comps
    mux = jnp.sum(gx) / cnt
    muy = jnp.sum(gy) / cnt
    muz = jnp.sum(gz) / cnt
    mxx = jnp.sum(gx * gx) / cnt
    myy = jnp.sum(gy * gy) / cnt
    mzz = jnp.sum(gz * gz) / cnt
    mxy = jnp.sum(gx * gy) / cnt
    mxz = jnp.sum(gx * gz) / cnt
    myz = jnp.sum(gy * gz) / cnt
    wx = W4[0:1, :]
    wy = W4[1:2, :]
    wz = W4[2:3, :]
    mean = mux * wx + muy * wy + muz * wz
    ep2 = (wx * wx * mxx + wy * wy * myy + wz * wz * mzz
           + 2.0 * (wx * wy * mxy + wx * wz * mxz + wy * wz * myz))
    var = ep2 - mean * mean
    ps = g_ref[...] / jnp.sqrt(var + EPS)
    pscale_ref[...] = ps
    pshift_ref[...] = b_ref[...] - mean * ps


def _pstats_call(gpq, w4, gamma, beta):
    return pl.pallas_call(
        _pstats_body,
        out_shape=[jax.ShapeDtypeStruct((1, OUT_C), jnp.float32)] * 2,
    )(gpq, w4, gamma, beta)


def _fuse_body(gf_ref, gp4_ref, w4_ref, pscale_ref, pshift_ref, out_ref):
    pe = lax.dot_general(gp4_ref[...], w4_ref[...], (((1,), (0,)), ((), ())),
                         preferred_element_type=jnp.float32)
    pe = jnp.maximum(pe * pscale_ref[...] + pshift_ref[...], 0.0)
    res = gf_ref[...] + pe
    mb = res.shape[0] // K
    out_ref[...] = jnp.max(res.reshape(mb, K, OUT_C), axis=1)


def _fuse_call(gf, gp4, w4, pscale, pshift):
    mb = 64
    grid = (B * M) // mb
    return pl.pallas_call(
        _fuse_body,
        grid=(grid,),
        in_specs=[
            pl.BlockSpec((mb * K, OUT_C), lambda i: (i, 0)),
            pl.BlockSpec((mb * K, 4), lambda i: (i, 0)),
            pl.BlockSpec((4, OUT_C), lambda i: (0, 0)),
            pl.BlockSpec((1, OUT_C), lambda i: (0, 0)),
            pl.BlockSpec((1, OUT_C), lambda i: (0, 0)),
        ],
        out_specs=pl.BlockSpec((mb, OUT_C), lambda i: (i, 0)),
        out_shape=jax.ShapeDtypeStruct((B * M, OUT_C), jnp.float32),
    )(gf, gp4, w4, pscale, pshift)


def kernel(points, features, x_w, x_gamma, x_beta, pos_w, pos_gamma, pos_beta):
    pts_t = jnp.transpose(points, (0, 2, 1))
    px, py, pz = pts_t[:, 0, :], pts_t[:, 1, :], pts_t[:, 2, :]

    oxp, oyp, ozp = _fps_call(px.reshape(B, _FR, _FC), py.reshape(B, _FR, _FC),
                              pz.reshape(B, _FR, _FC))
    ox = oxp.reshape(B, M)
    oy = oyp.reshape(B, M)
    oz = ozp.reshape(B, M)
    centroids = jnp.stack([ox, oy, oz], axis=-1)

    f2 = features.reshape(B * N, IN_C)
    scale, shift = _xstats_call(f2, x_w, x_gamma.reshape(-1, 1),
                                x_beta.reshape(-1, 1))
    feats = _feats_call(f2, x_w.T, scale.reshape(1, -1), shift.reshape(1, -1))

    gp4f, gf = _ballquery_call(ox, oy, oz, px, py, pz, feats)
    gp4 = gp4f.reshape(R, 4)

    w4 = jnp.concatenate([pos_w, jnp.zeros((OUT_C, 1), jnp.float32)], axis=1).T
    pscale, pshift = _pstats_call(gp4f.reshape(R * 4 // 128, 128), w4,
                                  pos_gamma.reshape(1, -1),
                                  pos_beta.reshape(1, -1))

    out = _fuse_call(gf, gp4, w4, pscale, pshift)
    return (centroids, out.reshape(B, M, OUT_C))


# speedup vs baseline: 1.0683x; 1.0683x over previous
import functools

import jax
import jax.numpy as jnp
from jax import lax
from jax.experimental import pallas as pl
from jax.experimental.pallas import tpu as pltpu
from jax.experimental.pallas import tpu_sc as plsc

B = 4
N = 4096
STRIDE = 4
RADIUS = 0.2
K = 32
IN_C = 64
OUT_C = 128
EPS = 1e-5
M = N // STRIDE
R = B * M * K

NC, NS, L = 2, 16, 16
NW = NC * NS
CPW = (B * M) // NW
RPW = R // NW
GCH = 128


_FR = 8
_FC = N // _FR
_OC = 128
_OR = M // _OC


def _fps_body(xs_ref, ys_ref, zs_ref, p3_ref, ox_ref, oy_ref, oz_ref):
    nidx = (lax.broadcasted_iota(jnp.int32, (_FR, _FC), 0) * _FC
            + lax.broadcasted_iota(jnp.int32, (_FR, _FC), 1))
    orow = lax.broadcasted_iota(jnp.int32, (_OR, _OC), 0)
    ocol = lax.broadcasted_iota(jnp.int32, (_OR, _OC), 1)

    Xs = [xs_ref[b] for b in range(B)]
    Ys = [ys_ref[b] for b in range(B)]
    Zs = [zs_ref[b] for b in range(B)]
    Ps = [p3_ref[b] for b in range(B)]

    sel0 = (orow == 0) & (ocol == 0)
    st0 = []
    for b in range(B):
        cx = Xs[b][0:1, 0:1]
        cy = Ys[b][0:1, 0:1]
        cz = Zs[b][0:1, 0:1]
        dis0 = jnp.full((_FR, _FC), 1e10, jnp.float32)
        st0.extend([dis0, cx, cy, cz,
                    jnp.where(sel0, cx, 0.0),
                    jnp.where(sel0, cy, 0.0),
                    jnp.where(sel0, cz, 0.0)])

    def body(i, carry):
        st = list(carry)
        sel = (orow == i // _OC) & (ocol == i % _OC)
        for b in range(B):
            dis, cx, cy, cz, ax, ay, az = st[7 * b:7 * b + 7]
            d = (Xs[b] - cx) ** 2 + (Ys[b] - cy) ** 2 + (Zs[b] - cz) ** 2
            dis = jnp.minimum(dis, d)
            v = jnp.max(jnp.max(dis, axis=1, keepdims=True), axis=0,
                        keepdims=True)
            cand = jnp.where(dis == v, nidx, N)
            idx = jnp.min(jnp.min(cand, axis=1, keepdims=True), axis=0,
                          keepdims=True)
            m1 = nidx == idx
            m3 = jnp.broadcast_to(m1[None], (3, _FR, _FC))
            s3 = jnp.sum(jnp.where(m3, Ps[b], 0.0), axis=(1, 2),
                         keepdims=True)
            ncx = s3[0]
            ncy = s3[1]
            ncz = s3[2]
            st[7 * b:7 * b + 7] = [dis, ncx, ncy, ncz,
                                   jnp.where(sel, ncx, ax),
                                   jnp.where(sel, ncy, ay),
                                   jnp.where(sel, ncz, az)]
        return tuple(st)

    st = lax.fori_loop(1, M, body, tuple(st0))
    for b in range(B):
        ox_ref[b] = st[7 * b + 4]
        oy_ref[b] = st[7 * b + 5]
        oz_ref[b] = st[7 * b + 6]


def _fps_call(xs, ys, zs):
    p3 = jnp.stack([xs, ys, zs], axis=1)
    return pl.pallas_call(
        _fps_body,
        out_shape=[jax.ShapeDtypeStruct((B, _OR, _OC), jnp.float32)] * 3,
    )(xs, ys, zs, p3)


def _xstats_body(f2_ref, xw_ref, g_ref, b_ref, scale_ref, shift_ref):
    F = f2_ref[...]
    W = xw_ref[...]
    cnt = float(B * N)
    G = lax.dot_general(F, F, (((0,), (0,)), ((), ())),
                        preferred_element_type=jnp.float32) / cnt
    mu = jnp.sum(F, axis=0, keepdims=True) / cnt
    mean = lax.dot_general(W, mu, (((1,), (1,)), ((), ())))
    T = lax.dot_general(W, G, (((1,), (0,)), ((), ())),
                        preferred_element_type=jnp.float32)
    ey2 = jnp.sum(T * W, axis=1, keepdims=True)
    var = ey2 - mean * mean
    sc = g_ref[...] / jnp.sqrt(var + EPS)
    scale_ref[...] = sc
    shift_ref[...] = b_ref[...] - mean * sc


def _xstats_call(f2, x_w, gamma, beta):
    return pl.pallas_call(
        _xstats_body,
        out_shape=[jax.ShapeDtypeStruct((OUT_C, 1), jnp.float32)] * 2,
    )(f2, x_w, gamma, beta)


def _feats_body(f2_ref, wt_ref, scale_ref, shift_ref, out_ref):
    y = lax.dot_general(f2_ref[...], wt_ref[...], (((1,), (0,)), ((), ())),
                        preferred_element_type=jnp.float32)
    out_ref[...] = jnp.maximum(y * scale_ref[...] + shift_ref[...], 0.0)


def _feats_call(f2, x_wt, scale, shift):
    blk = 2048
    grid = (B * N) // blk
    return pl.pallas_call(
        _feats_body,
        grid=(grid,),
        in_specs=[
            pl.BlockSpec((blk, IN_C), lambda i: (i, 0)),
            pl.BlockSpec((IN_C, OUT_C), lambda i: (0, 0)),
            pl.BlockSpec((1, OUT_C), lambda i: (0, 0)),
            pl.BlockSpec((1, OUT_C), lambda i: (0, 0)),
        ],
        out_specs=pl.BlockSpec((blk, OUT_C), lambda i: (i, 0)),
        out_shape=jax.ShapeDtypeStruct((B * N, OUT_C), jnp.float32),
    )(f2, x_wt, scale, shift)


def _ballquery_call(cx, cy, cz, px, py, pz):
    mesh = plsc.VectorSubcoreMesh(core_axis_name="c", subcore_axis_name="s")

    @functools.partial(
        pl.kernel,
        mesh=mesh,
        compiler_params=pltpu.CompilerParams(needs_layout_passes=False),
        out_type=[
            jax.ShapeDtypeStruct((R,), jnp.int32),
            jax.ShapeDtypeStruct((R * 4,), jnp.float32),
        ],
        scratch_types=[
            pltpu.VMEM((N,), jnp.float32),
            pltpu.VMEM((N,), jnp.float32),
            pltpu.VMEM((N,), jnp.float32),
            pltpu.VMEM((CPW,), jnp.float32),
            pltpu.VMEM((CPW,), jnp.float32),
            pltpu.VMEM((CPW,), jnp.float32),
            pltpu.VMEM((256,), jnp.int32),
            pltpu.VMEM((CPW * K,), jnp.int32),
            pltpu.VMEM((CPW * K * 4,), jnp.float32),
        ],
    )
    def k(cxh, cyh, czh, pxh, pyh, pzh, idxh, gp4h,
          pxv, pyv, pzv, cxv, cyv, czv, hitv, idxv, gpv):
        wid = lax.axis_index("s") * NC + lax.axis_index("c")
        g0 = wid * CPW
        bi = g0 // M
        off = g0 - bi * M
        pltpu.sync_copy(pxh.at[bi], pxv)
        pltpu.sync_copy(pyh.at[bi], pyv)
        pltpu.sync_copy(pzh.at[bi], pzv)
        pltpu.sync_copy(cxh.at[bi, pl.ds(off, CPW)], cxv)
        pltpu.sync_copy(cyh.at[bi, pl.ds(off, CPW)], cyv)
        pltpu.sync_copy(czh.at[bi, pl.ds(off, CPW)], czv)

        lane = lax.iota(jnp.int32, 16)
        zero16 = jnp.zeros((16,), jnp.int32)
        col0 = zero16
        col1 = zero16 + 1
        col2 = zero16 + 2
        col3 = zero16 + 3
        zf16 = jnp.zeros((16,), jnp.float32)
        r2 = RADIUS * RADIUS
        gbase = bi * N

        czero = jnp.zeros((16,), jnp.int32)

        NG = 4

        def per_quad(cp, _):
            ia = jnp.full((16,), NG * cp, jnp.int32)
            cents = [(plsc.load_gather(cxv, [ia + t]),
                      plsc.load_gather(cyv, [ia + t]),
                      plsc.load_gather(czv, [ia + t])) for t in range(NG)]

            def cond(st):
                j = st[0]
                cs = st[1:]
                lo = jnp.minimum(jnp.minimum(cs[0], cs[1]),
                                 jnp.minimum(cs[2], cs[3]))
                return (j < N // 16) & (jnp.max(lo) < K)

            def body(st):
                j = st[0]
                cnts = list(st[1:])
                for u in range(2):
                    base = (j + u) * 16
                    x16 = pxv[pl.ds(base, 16)]
                    y16 = pyv[pl.ds(base, 16)]
                    z16 = pzv[pl.ds(base, 16)]
                    gidx = base + lane
                    for t in range(NG):
                        cxs, cys, czs = cents[t]
                        dx = x16 - cxs
                        dy = y16 - cys
                        dz = z16 - czs
                        d2 = dx * dx + dy * dy + dz * dz
                        msk = d2 <= r2
                        mi = msk.astype(jnp.int32)
                        pos = cnts[t] + plsc.cumsum(mi) - mi
                        ok = msk & (pos < 64)
                        plsc.store_scatter(hitv, [pos + 64 * t], gidx, mask=ok)
                        cnts[t] = cnts[t] + plsc.all_reduce_population_count(msk)
                return (j + 2, *cnts)

            st = lax.while_loop(cond, body, (0, czero, czero, czero, czero))
            for t in range(NG):
                cxs, cys, czs = cents[t]
                cnt = jnp.minimum(jnp.max(st[1 + t]), K)
                hb = 64 * t
                first = plsc.load_gather(hitv, [jnp.full((16,), hb, jnp.int32)])
                o0 = jnp.where(lane < cnt, hitv[pl.ds(hb, 16)], first)
                o1 = jnp.where(lane + 16 < cnt, hitv[pl.ds(hb + 16, 16)], first)
                c = NG * cp + t
                rows0 = c * K + lane
                rows1 = rows0 + 16
                for o, rows in ((o0, rows0), (o1, rows1)):
                    gx = (plsc.load_gather(pxv, [o]) - cxs) / RADIUS
                    gy = (plsc.load_gather(pyv, [o]) - cys) / RADIUS
                    gz = (plsc.load_gather(pzv, [o]) - czs) / RADIUS
                    flat = rows * 4
                    plsc.store_scatter(gpv, [flat], gx)
                    plsc.store_scatter(gpv, [flat + 1], gy)
                    plsc.store_scatter(gpv, [flat + 2], gz)
                    plsc.store_scatter(gpv, [flat + 3], zf16)
                idxv[pl.ds(c * K, 16)] = o0 + gbase
                idxv[pl.ds(c * K + 16, 16)] = o1 + gbase
            return 0

        lax.fori_loop(0, CPW // NG, per_quad, 0)
        pltpu.sync_copy(idxv, idxh.at[pl.ds(g0 * K, CPW * K)])
        pltpu.sync_copy(gpv, gp4h.at[pl.ds(g0 * K * 4, CPW * K * 4)])

    return k(cx, cy, cz, px, py, pz)


def _gather_call(feats, idx):
    mesh = plsc.VectorSubcoreMesh(core_axis_name="c", subcore_axis_name="s")

    @functools.partial(
        pl.kernel,
        mesh=mesh,
        compiler_params=pltpu.CompilerParams(needs_layout_passes=False),
        out_type=jax.ShapeDtypeStruct((R, OUT_C), jnp.float32),
        scratch_types=[
            pltpu.VMEM((GCH,), jnp.int32),
            pltpu.VMEM((GCH,), jnp.int32),
            pltpu.VMEM((GCH, OUT_C), jnp.float32),
            pltpu.VMEM((GCH, OUT_C), jnp.float32),
            pltpu.SemaphoreType.DMA,
            pltpu.SemaphoreType.DMA,
        ],
    )
    def k(fh, ih, oh, iv0, iv1, bv0, bv1, sem0, sem1):
        wid = lax.axis_index("s") * NC + lax.axis_index("c")
        r0 = wid * RPW
        nch = RPW // GCH
        ivs = (iv0, iv1)
        bvs = (bv0, bv1)
        sems = (sem0, sem1)

        pltpu.sync_copy(ih.at[pl.ds(r0, GCH)], iv0)
        pltpu.async_copy(fh.at[iv0], bv0, sem0)

        def outer(g2, _):
            g = g2 * 2
            for par in range(2):
                t = g + par
                nxt = t + 1
                np_ = (par + 1) % 2
                iv_n, bv_n, sem_n = ivs[np_], bvs[np_], sems[np_]
                iv_c, bv_c, sem_c = ivs[par], bvs[par], sems[par]

                @pl.when(nxt < nch)
                def _():
                    pltpu.sync_copy(ih.at[pl.ds(r0 + nxt * GCH, GCH)], iv_n)
                    pltpu.async_copy(fh.at[iv_n], bv_n, sem_n)

                pltpu.make_async_copy(fh.at[iv_c], bv_c, sem_c).wait()
                pltpu.sync_copy(bv_c, oh.at[pl.ds(r0 + t * GCH, GCH)])
            return 0

        lax.fori_loop(0, nch // 2, outer, 0)

    return k(feats, idx)


def _pstats_body(gpq_ref, w4_ref, g_ref, b_ref, pscale_ref, pshift_ref):
    A = gpq_ref[...]
    W4 = w4_ref[...]
    cnt = float(R)
    row = lax.broadcasted_iota(jnp.int32, (128, K), 0)
    col = lax.broadcasted_iota(jnp.int32, (128, K), 1)
    comps = []
    for c in range(3):
        S = (row == col * 4 + c).astype(jnp.float32)
        comps.append(lax.dot_general(A, S, (((1,), (0,)), ((), ())),
                                     preferred_element_type=jnp.float32))
    gx, gy, gz = comps
    mux = jnp.sum(gx) / cnt
    muy = jnp.sum(gy) / cnt
    muz = jnp.sum(gz) / cnt
    mxx = jnp.sum(gx * gx) / cnt
    myy = jnp.sum(gy * gy) / cnt
    mzz = jnp.sum(gz * gz) / cnt
    mxy = jnp.sum(gx * gy) / cnt
    mxz = jnp.sum(gx * gz) / cnt
    myz = jnp.sum(gy * gz) / cnt
    wx = W4[0:1, :]
    wy = W4[1:2, :]
    wz = W4[2:3, :]
    mean = mux * wx + muy * wy + muz * wz
    ep2 = (wx * wx * mxx + wy * wy * myy + wz * wz * mzz
           + 2.0 * (wx * wy * mxy + wx * wz * mxz + wy * wz * myz))
    var = ep2 - mean * mean
    ps = g_ref[...] / jnp.sqrt(var + EPS)
    pscale_ref[...] = ps
    pshift_ref[...] = b_ref[...] - mean * ps


def _pstats_call(gpq, w4, gamma, beta):
    return pl.pallas_call(
        _pstats_body,
        out_shape=[jax.ShapeDtypeStruct((1, OUT_C), jnp.float32)] * 2,
    )(gpq, w4, gamma, beta)


def _fuse_body(gf_ref, gp4_ref, w4_ref, pscale_ref, pshift_ref, out_ref):
    pe = lax.dot_general(gp4_ref[...], w4_ref[...], (((1,), (0,)), ((), ())),
                         preferred_element_type=jnp.float32)
    pe = jnp.maximum(pe * pscale_ref[...] + pshift_ref[...], 0.0)
    res = gf_ref[...] + pe
    mb = res.shape[0] // K
    out_ref[...] = jnp.max(res.reshape(mb, K, OUT_C), axis=1)


def _fuse_call(gf, gp4, w4, pscale, pshift):
    mb = 64
    grid = (B * M) // mb
    return pl.pallas_call(
        _fuse_body,
        grid=(grid,),
        in_specs=[
            pl.BlockSpec((mb * K, OUT_C), lambda i: (i, 0)),
            pl.BlockSpec((mb * K, 4), lambda i: (i, 0)),
            pl.BlockSpec((4, OUT_C), lambda i: (0, 0)),
            pl.BlockSpec((1, OUT_C), lambda i: (0, 0)),
            pl.BlockSpec((1, OUT_C), lambda i: (0, 0)),
        ],
        out_specs=pl.BlockSpec((mb, OUT_C), lambda i: (i, 0)),
        out_shape=jax.ShapeDtypeStruct((B * M, OUT_C), jnp.float32),
    )(gf, gp4, w4, pscale, pshift)


def kernel(points, features, x_w, x_gamma, x_beta, pos_w, pos_gamma, pos_beta):
    pts_t = jnp.transpose(points, (0, 2, 1))
    px, py, pz = pts_t[:, 0, :], pts_t[:, 1, :], pts_t[:, 2, :]

    oxp, oyp, ozp = _fps_call(px.reshape(B, _FR, _FC), py.reshape(B, _FR, _FC),
                              pz.reshape(B, _FR, _FC))
    ox = oxp.reshape(B, M)
    oy = oyp.reshape(B, M)
    oz = ozp.reshape(B, M)
    centroids = jnp.stack([ox, oy, oz], axis=-1)

    f2 = features.reshape(B * N, IN_C)
    scale, shift = _xstats_call(f2, x_w, x_gamma.reshape(-1, 1),
                                x_beta.reshape(-1, 1))
    feats = _feats_call(f2, x_w.T, scale.reshape(1, -1), shift.reshape(1, -1))

    idx, gp4f = _ballquery_call(ox, oy, oz, px, py, pz)
    gp4 = gp4f.reshape(R, 4)

    w4 = jnp.concatenate([pos_w, jnp.zeros((OUT_C, 1), jnp.float32)], axis=1).T
    pscale, pshift = _pstats_call(gp4f.reshape(R * 4 // 128, 128), w4,
                                  pos_gamma.reshape(1, -1),
                                  pos_beta.reshape(1, -1))

    gf = _gather_call(feats, idx)
    out = _fuse_call(gf, gp4, w4, pscale, pshift)
    return (centroids, out.reshape(B, M, OUT_C))


# speedup vs baseline: 1.0840x; 1.0147x over previous
import functools

import jax
import jax.numpy as jnp
from jax import lax
from jax.experimental import pallas as pl
from jax.experimental.pallas import tpu as pltpu
from jax.experimental.pallas import tpu_sc as plsc

B = 4
N = 4096
STRIDE = 4
RADIUS = 0.2
K = 32
IN_C = 64
OUT_C = 128
EPS = 1e-5
M = N // STRIDE
R = B * M * K

NC, NS, L = 2, 16, 16
NW = NC * NS
CPW = (B * M) // NW
RPW = R // NW
GCH = 128


_FR = 8
_FC = N // _FR
_OC = 128
_OR = M // _OC


def _fps_body(xs_ref, ys_ref, zs_ref, p3_ref, ox_ref, oy_ref, oz_ref):
    nidx = (lax.broadcasted_iota(jnp.int32, (_FR, _FC), 0) * _FC
            + lax.broadcasted_iota(jnp.int32, (_FR, _FC), 1))
    orow = lax.broadcasted_iota(jnp.int32, (_OR, _OC), 0)
    ocol = lax.broadcasted_iota(jnp.int32, (_OR, _OC), 1)

    Xs = [xs_ref[b] for b in range(B)]
    Ys = [ys_ref[b] for b in range(B)]
    Zs = [zs_ref[b] for b in range(B)]
    Ps = [p3_ref[b] for b in range(B)]

    sel0 = (orow == 0) & (ocol == 0)
    st0 = []
    for b in range(B):
        cx = Xs[b][0:1, 0:1]
        cy = Ys[b][0:1, 0:1]
        cz = Zs[b][0:1, 0:1]
        dis0 = jnp.full((_FR, _FC), 1e10, jnp.float32)
        st0.extend([dis0, cx, cy, cz,
                    jnp.where(sel0, cx, 0.0),
                    jnp.where(sel0, cy, 0.0),
                    jnp.where(sel0, cz, 0.0)])

    def body(i, carry):
        st = list(carry)
        sel = (orow == i // _OC) & (ocol == i % _OC)
        for b in range(B):
            dis, cx, cy, cz, ax, ay, az = st[7 * b:7 * b + 7]
            d = (Xs[b] - cx) ** 2 + (Ys[b] - cy) ** 2 + (Zs[b] - cz) ** 2
            dis = jnp.minimum(dis, d)
            v = jnp.max(jnp.max(dis, axis=1, keepdims=True), axis=0,
                        keepdims=True)
            cand = jnp.where(dis == v, nidx, N)
            idx = jnp.min(jnp.min(cand, axis=1, keepdims=True), axis=0,
                          keepdims=True)
            m1 = nidx == idx
            m3 = jnp.broadcast_to(m1[None], (3, _FR, _FC))
            s3 = jnp.sum(jnp.where(m3, Ps[b], 0.0), axis=(1, 2),
                         keepdims=True)
            ncx = s3[0]
            ncy = s3[1]
            ncz = s3[2]
            st[7 * b:7 * b + 7] = [dis, ncx, ncy, ncz,
                                   jnp.where(sel, ncx, ax),
                                   jnp.where(sel, ncy, ay),
                                   jnp.where(sel, ncz, az)]
        return tuple(st)

    st = lax.fori_loop(1, M, body, tuple(st0))
    for b in range(B):
        ox_ref[b] = st[7 * b + 4]
        oy_ref[b] = st[7 * b + 5]
        oz_ref[b] = st[7 * b + 6]


def _fps_call(xs, ys, zs):
    p3 = jnp.stack([xs, ys, zs], axis=1)
    return pl.pallas_call(
        _fps_body,
        out_shape=[jax.ShapeDtypeStruct((B, _OR, _OC), jnp.float32)] * 3,
    )(xs, ys, zs, p3)


def _xstats_body(f2_ref, xw_ref, g_ref, b_ref, scale_ref, shift_ref):
    F = f2_ref[...]
    W = xw_ref[...]
    cnt = float(B * N)
    G = lax.dot_general(F, F, (((0,), (0,)), ((), ())),
                        preferred_element_type=jnp.float32) / cnt
    mu = jnp.sum(F, axis=0, keepdims=True) / cnt
    mean = lax.dot_general(W, mu, (((1,), (1,)), ((), ())))
    T = lax.dot_general(W, G, (((1,), (0,)), ((), ())),
                        preferred_element_type=jnp.float32)
    ey2 = jnp.sum(T * W, axis=1, keepdims=True)
    var = ey2 - mean * mean
    sc = g_ref[...] / jnp.sqrt(var + EPS)
    scale_ref[...] = sc
    shift_ref[...] = b_ref[...] - mean * sc


def _xstats_call(f2, x_w, gamma, beta):
    return pl.pallas_call(
        _xstats_body,
        out_shape=[jax.ShapeDtypeStruct((OUT_C, 1), jnp.float32)] * 2,
    )(f2, x_w, gamma, beta)


def _feats_body(f2_ref, wt_ref, scale_ref, shift_ref, out_ref):
    y = lax.dot_general(f2_ref[...], wt_ref[...], (((1,), (0,)), ((), ())),
                        preferred_element_type=jnp.float32)
    out_ref[...] = jnp.maximum(y * scale_ref[...] + shift_ref[...], 0.0)


def _feats_call(f2, x_wt, scale, shift):
    blk = 2048
    grid = (B * N) // blk
    return pl.pallas_call(
        _feats_body,
        grid=(grid,),
        in_specs=[
            pl.BlockSpec((blk, IN_C), lambda i: (i, 0)),
            pl.BlockSpec((IN_C, OUT_C), lambda i: (0, 0)),
            pl.BlockSpec((1, OUT_C), lambda i: (0, 0)),
            pl.BlockSpec((1, OUT_C), lambda i: (0, 0)),
        ],
        out_specs=pl.BlockSpec((blk, OUT_C), lambda i: (i, 0)),
        out_shape=jax.ShapeDtypeStruct((B * N, OUT_C), jnp.float32),
    )(f2, x_wt, scale, shift)


def _ballquery_call(cx, cy, cz, px, py, pz):
    mesh = plsc.VectorSubcoreMesh(core_axis_name="c", subcore_axis_name="s")

    @functools.partial(
        pl.kernel,
        mesh=mesh,
        compiler_params=pltpu.CompilerParams(needs_layout_passes=False),
        out_type=[
            jax.ShapeDtypeStruct((R,), jnp.int32),
            jax.ShapeDtypeStruct((R * 4,), jnp.float32),
        ],
        scratch_types=[
            pltpu.VMEM((N,), jnp.float32),
            pltpu.VMEM((N,), jnp.float32),
            pltpu.VMEM((N,), jnp.float32),
            pltpu.VMEM((CPW,), jnp.float32),
            pltpu.VMEM((CPW,), jnp.float32),
            pltpu.VMEM((CPW,), jnp.float32),
            pltpu.VMEM((512,), jnp.int32),
            pltpu.VMEM((CPW * K,), jnp.int32),
            pltpu.VMEM((CPW * K * 4,), jnp.float32),
        ],
    )
    def k(cxh, cyh, czh, pxh, pyh, pzh, idxh, gp4h,
          pxv, pyv, pzv, cxv, cyv, czv, hitv, idxv, gpv):
        wid = lax.axis_index("s") * NC + lax.axis_index("c")
        g0 = wid * CPW
        bi = g0 // M
        off = g0 - bi * M
        pltpu.sync_copy(pxh.at[bi], pxv)
        pltpu.sync_copy(pyh.at[bi], pyv)
        pltpu.sync_copy(pzh.at[bi], pzv)
        pltpu.sync_copy(cxh.at[bi, pl.ds(off, CPW)], cxv)
        pltpu.sync_copy(cyh.at[bi, pl.ds(off, CPW)], cyv)
        pltpu.sync_copy(czh.at[bi, pl.ds(off, CPW)], czv)

        lane = lax.iota(jnp.int32, 16)
        zero16 = jnp.zeros((16,), jnp.int32)
        col0 = zero16
        col1 = zero16 + 1
        col2 = zero16 + 2
        col3 = zero16 + 3
        zf16 = jnp.zeros((16,), jnp.float32)
        r2 = RADIUS * RADIUS
        gbase = bi * N

        czero = jnp.zeros((16,), jnp.int32)

        NG = 8

        def per_quad(cp, _):
            ia = jnp.full((16,), NG * cp, jnp.int32)
            cents = [(plsc.load_gather(cxv, [ia + t]),
                      plsc.load_gather(cyv, [ia + t]),
                      plsc.load_gather(czv, [ia + t])) for t in range(NG)]

            def cond(st):
                j = st[0]
                cs = list(st[1:])
                while len(cs) > 1:
                    cs = [jnp.minimum(cs[2 * i], cs[2 * i + 1])
                          for i in range(len(cs) // 2)]
                return (j < N // 16) & (jnp.max(cs[0]) < K)

            def body(st):
                j = st[0]
                cnts = list(st[1:])
                for u in range(2):
                    base = (j + u) * 16
                    x16 = pxv[pl.ds(base, 16)]
                    y16 = pyv[pl.ds(base, 16)]
                    z16 = pzv[pl.ds(base, 16)]
                    gidx = base + lane
                    for t in range(NG):
                        cxs, cys, czs = cents[t]
                        dx = x16 - cxs
                        dy = y16 - cys
                        dz = z16 - czs
                        d2 = dx * dx + dy * dy + dz * dz
                        msk = d2 <= r2
                        mi = msk.astype(jnp.int32)
                        pos = cnts[t] + plsc.cumsum(mi) - mi
                        ok = msk & (pos < 64)
                        plsc.store_scatter(hitv, [pos + 64 * t], gidx, mask=ok)
                        cnts[t] = cnts[t] + plsc.all_reduce_population_count(msk)
                return (j + 2, *cnts)

            st = lax.while_loop(cond, body, (0,) + (czero,) * NG)
            for t in range(NG):
                cxs, cys, czs = cents[t]
                cnt = jnp.minimum(jnp.max(st[1 + t]), K)
                hb = 64 * t
                first = plsc.load_gather(hitv, [jnp.full((16,), hb, jnp.int32)])
                o0 = jnp.where(lane < cnt, hitv[pl.ds(hb, 16)], first)
                o1 = jnp.where(lane + 16 < cnt, hitv[pl.ds(hb + 16, 16)], first)
                c = NG * cp + t
                rows0 = c * K + lane
                rows1 = rows0 + 16
                for o, rows in ((o0, rows0), (o1, rows1)):
                    gx = (plsc.load_gather(pxv, [o]) - cxs) / RADIUS
                    gy = (plsc.load_gather(pyv, [o]) - cys) / RADIUS
                    gz = (plsc.load_gather(pzv, [o]) - czs) / RADIUS
                    flat = rows * 4
                    plsc.store_scatter(gpv, [flat], gx)
                    plsc.store_scatter(gpv, [flat + 1], gy)
                    plsc.store_scatter(gpv, [flat + 2], gz)
                    plsc.store_scatter(gpv, [flat + 3], zf16)
                idxv[pl.ds(c * K, 16)] = o0 + gbase
                idxv[pl.ds(c * K + 16, 16)] = o1 + gbase
            return 0

        lax.fori_loop(0, CPW // NG, per_quad, 0)
        pltpu.sync_copy(idxv, idxh.at[pl.ds(g0 * K, CPW * K)])
        pltpu.sync_copy(gpv, gp4h.at[pl.ds(g0 * K * 4, CPW * K * 4)])

    return k(cx, cy, cz, px, py, pz)


def _gather_call(feats, idx):
    mesh = plsc.VectorSubcoreMesh(core_axis_name="c", subcore_axis_name="s")

    @functools.partial(
        pl.kernel,
        mesh=mesh,
        compiler_params=pltpu.CompilerParams(needs_layout_passes=False),
        out_type=jax.ShapeDtypeStruct((R, OUT_C), jnp.float32),
        scratch_types=[
            pltpu.VMEM((GCH,), jnp.int32),
            pltpu.VMEM((GCH,), jnp.int32),
            pltpu.VMEM((GCH, OUT_C), jnp.float32),
            pltpu.VMEM((GCH, OUT_C), jnp.float32),
            pltpu.SemaphoreType.DMA,
            pltpu.SemaphoreType.DMA,
        ],
    )
    def k(fh, ih, oh, iv0, iv1, bv0, bv1, sem0, sem1):
        wid = lax.axis_index("s") * NC + lax.axis_index("c")
        r0 = wid * RPW
        nch = RPW // GCH
        ivs = (iv0, iv1)
        bvs = (bv0, bv1)
        sems = (sem0, sem1)

        pltpu.sync_copy(ih.at[pl.ds(r0, GCH)], iv0)
        pltpu.async_copy(fh.at[iv0], bv0, sem0)

        def outer(g2, _):
            g = g2 * 2
            for par in range(2):
                t = g + par
                nxt = t + 1
                np_ = (par + 1) % 2
                iv_n, bv_n, sem_n = ivs[np_], bvs[np_], sems[np_]
                iv_c, bv_c, sem_c = ivs[par], bvs[par], sems[par]

                @pl.when(nxt < nch)
                def _():
                    pltpu.sync_copy(ih.at[pl.ds(r0 + nxt * GCH, GCH)], iv_n)
                    pltpu.async_copy(fh.at[iv_n], bv_n, sem_n)

                pltpu.make_async_copy(fh.at[iv_c], bv_c, sem_c).wait()
                pltpu.sync_copy(bv_c, oh.at[pl.ds(r0 + t * GCH, GCH)])
            return 0

        lax.fori_loop(0, nch // 2, outer, 0)

    return k(feats, idx)


def _pstats_body(gpq_ref, w4_ref, g_ref, b_ref, pscale_ref, pshift_ref):
    A = gpq_ref[...]
    W4 = w4_ref[...]
    cnt = float(R)
    row = lax.broadcasted_iota(jnp.int32, (128, K), 0)
    col = lax.broadcasted_iota(jnp.int32, (128, K), 1)
    comps = []
    for c in range(3):
        S = (row == col * 4 + c).astype(jnp.float32)
        comps.append(lax.dot_general(A, S, (((1,), (0,)), ((), ())),
                                     preferred_element_type=jnp.float32))
    gx, gy, gz = comps
    mux = jnp.sum(gx) / cnt
    muy = jnp.sum(gy) / cnt
    muz = jnp.sum(gz) / cnt
    mxx = jnp.sum(gx * gx) / cnt
    myy = jnp.sum(gy * gy) / cnt
    mzz = jnp.sum(gz * gz) / cnt
    mxy = jnp.sum(gx * gy) / cnt
    mxz = jnp.sum(gx * gz) / cnt
    myz = jnp.sum(gy * gz) / cnt
    wx = W4[0:1, :]
    wy = W4[1:2, :]
    wz = W4[2:3, :]
    mean = mux * wx + muy * wy + muz * wz
    ep2 = (wx * wx * mxx + wy * wy * myy + wz * wz * mzz
           + 2.0 * (wx * wy * mxy + wx * wz * mxz + wy * wz * myz))
    var = ep2 - mean * mean
    ps = g_ref[...] / jnp.sqrt(var + EPS)
    pscale_ref[...] = ps
    pshift_ref[...] = b_ref[...] - mean * ps


def _pstats_call(gpq, w4, gamma, beta):
    return pl.pallas_call(
        _pstats_body,
        out_shape=[jax.ShapeDtypeStruct((1, OUT_C), jnp.float32)] * 2,
    )(gpq, w4, gamma, beta)


def _fuse_body(gf_ref, gp4_ref, w4_ref, pscale_ref, pshift_ref, out_ref):
    pe = lax.dot_general(gp4_ref[...], w4_ref[...], (((1,), (0,)), ((), ())),
                         preferred_element_type=jnp.float32)
    pe = jnp.maximum(pe * pscale_ref[...] + pshift_ref[...], 0.0)
    res = gf_ref[...] + pe
    mb = res.shape[0] // K
    out_ref[...] = jnp.max(res.reshape(mb, K, OUT_C), axis=1)


def _fuse_call(gf, gp4, w4, pscale, pshift):
    mb = 64
    grid = (B * M) // mb
    return pl.pallas_call(
        _fuse_body,
        grid=(grid,),
        in_specs=[
            pl.BlockSpec((mb * K, OUT_C), lambda i: (i, 0)),
            pl.BlockSpec((mb * K, 4), lambda i: (i, 0)),
            pl.BlockSpec((4, OUT_C), lambda i: (0, 0)),
            pl.BlockSpec((1, OUT_C), lambda i: (0, 0)),
            pl.BlockSpec((1, OUT_C), lambda i: (0, 0)),
        ],
        out_specs=pl.BlockSpec((mb, OUT_C), lambda i: (i, 0)),
        out_shape=jax.ShapeDtypeStruct((B * M, OUT_C), jnp.float32),
    )(gf, gp4, w4, pscale, pshift)


def kernel(points, features, x_w, x_gamma, x_beta, pos_w, pos_gamma, pos_beta):
    pts_t = jnp.transpose(points, (0, 2, 1))
    px, py, pz = pts_t[:, 0, :], pts_t[:, 1, :], pts_t[:, 2, :]

    oxp, oyp, ozp = _fps_call(px.reshape(B, _FR, _FC), py.reshape(B, _FR, _FC),
                              pz.reshape(B, _FR, _FC))
    ox = oxp.reshape(B, M)
    oy = oyp.reshape(B, M)
    oz = ozp.reshape(B, M)
    centroids = jnp.stack([ox, oy, oz], axis=-1)

    f2 = features.reshape(B * N, IN_C)
    scale, shift = _xstats_call(f2, x_w, x_gamma.reshape(-1, 1),
                                x_beta.reshape(-1, 1))
    feats = _feats_call(f2, x_w.T, scale.reshape(1, -1), shift.reshape(1, -1))

    idx, gp4f = _ballquery_call(ox, oy, oz, px, py, pz)
    gp4 = gp4f.reshape(R, 4)

    w4 = jnp.concatenate([pos_w, jnp.zeros((OUT_C, 1), jnp.float32)], axis=1).T
    pscale, pshift = _pstats_call(gp4f.reshape(R * 4 // 128, 128), w4,
                                  pos_gamma.reshape(1, -1),
                                  pos_beta.reshape(1, -1))

    gf = _gather_call(feats, idx)
    out = _fuse_call(gf, gp4, w4, pscale, pshift)
    return (centroids, out.reshape(B, M, OUT_C))
